# Initial kernel scaffold; baseline (speedup 1.0000x reference)
import jax
import jax.numpy as jnp
from jax.experimental import pallas as pl


def kernel(query_length, key_length, bias_table):
    raise NotImplementedError("write your pallas kernel here")


# speedup vs baseline: 41.3780x; 41.3780x over previous
import functools
import math

import numpy as np
import jax
import jax.numpy as jnp
from jax import lax
from jax.experimental import pallas as pl
from jax.experimental.pallas import tpu as pltpu
from jax.experimental.pallas import tpu_sc as plsc

Q_LEN = 2048
K_LEN = 2048
NUM_HEADS = 16
NUM_BUCKETS = 32
MAX_DISTANCE = 128

NUM_SHIFTS = 8
LINE_LEN = 4112
CHUNKS = LINE_LEN // 16
ROWS_PER_WORKER = Q_LEN // 2
DMA_GROUP = 8


def _bucket_shift_table() -> np.ndarray:
    d = np.arange(-(Q_LEN - 1), K_LEN, dtype=np.int32)
    half = NUM_BUCKETS // 2
    buckets = (d > 0).astype(np.int32) * half
    rp = np.abs(d)
    max_exact = half // 2
    is_small = rp < max_exact
    safe_rp = np.maximum(rp, 1).astype(np.float32)
    large = max_exact + (
        np.log(safe_rp / np.float32(max_exact))
        / np.float32(math.log(MAX_DISTANCE / max_exact))
        * np.float32(half - max_exact)
    ).astype(np.int32)
    large = np.minimum(large, half - 1)
    line = buckets + np.where(is_small, rp, large)

    j = np.arange(LINE_LEN, dtype=np.int32)
    idx = np.minimum(j[None, :] + np.arange(NUM_SHIFTS, dtype=np.int32)[:, None],
                     line.shape[0] - 1)
    return line[idx].astype(np.int32)


_BUCKET_SHIFT = _bucket_shift_table()


def _sc_body(table_hbm, bidx_hbm, out_hbm, table_v, *rest):
    bidx_v = rest[:NUM_SHIFTS]
    lines_v = rest[NUM_SHIFTS:2 * NUM_SHIFTS]
    sem_out = rest[2 * NUM_SHIFTS]

    h = lax.axis_index("s")
    half = lax.axis_index("c")

    pltpu.sync_copy(table_hbm, table_v)
    for s in range(NUM_SHIFTS):
        pltpu.sync_copy(bidx_hbm.at[s], bidx_v[s])

    h_vec = jnp.full((16,), h, dtype=jnp.int32)
    for s in range(NUM_SHIFTS):
        def chunk(i, carry, s=s):
            bv = bidx_v[s][pl.ds(i * 16, 16)]
            lines_v[s][pl.ds(i * 16, 16)] = plsc.load_gather(table_v, [bv, h_vec])
            return carry
        lax.fori_loop(0, CHUNKS, chunk, 0)

    q0 = half * ROWS_PER_WORKER
    def rows(g, carry):
        qb = q0 + g * DMA_GROUP
        copies = []
        for j in range(DMA_GROUP):
            q = qb + j
            s = (Q_LEN - 1 - j) % NUM_SHIFTS
            base = (Q_LEN - 1 - q) - s
            copies.append(pltpu.async_copy(
                lines_v[s].at[pl.ds(base, K_LEN)],
                out_hbm.at[h, q],
                sem_out,
            ))
        for c in copies:
            c.wait()
        return carry
    lax.fori_loop(0, ROWS_PER_WORKER // DMA_GROUP, rows, 0)


@functools.partial(jax.jit, static_argnums=())
def _bias_sc(bias_table, bucket_shift):
    kern = pl.kernel(
        _sc_body,
        out_type=jax.ShapeDtypeStruct((NUM_HEADS, Q_LEN, K_LEN), jnp.float32),
        mesh=plsc.VectorSubcoreMesh(core_axis_name="c", subcore_axis_name="s"),
        scratch_types=(
            [pltpu.VMEM((NUM_BUCKETS, NUM_HEADS), jnp.float32)]
            + [pltpu.VMEM((LINE_LEN,), jnp.int32) for _ in range(NUM_SHIFTS)]
            + [pltpu.VMEM((LINE_LEN,), jnp.float32) for _ in range(NUM_SHIFTS)]
            + [pltpu.SemaphoreType.DMA]
        ),
        compiler_params=pltpu.CompilerParams(
            needs_layout_passes=False, use_tc_tiling_on_sc=False
        ),
    )
    return kern(bias_table, bucket_shift)


def kernel(query_length, key_length, bias_table):
    del query_length, key_length
    out = _bias_sc(bias_table, jnp.asarray(_BUCKET_SHIFT))
    return out[None]


# speedup vs baseline: 41.8315x; 1.0110x over previous
import functools
import math

import numpy as np
import jax
import jax.numpy as jnp
from jax import lax
from jax.experimental import pallas as pl
from jax.experimental.pallas import tpu as pltpu
from jax.experimental.pallas import tpu_sc as plsc

Q_LEN = 2048
K_LEN = 2048
NUM_HEADS = 16
NUM_BUCKETS = 32
MAX_DISTANCE = 128

NUM_SHIFTS = 8
LINE_LEN = 4112
CHUNKS = LINE_LEN // 16
ROWS_PER_WORKER = Q_LEN // 2
DMA_GROUP = 16


def _bucket_shift_table() -> np.ndarray:
    d = np.arange(-(Q_LEN - 1), K_LEN, dtype=np.int32)
    half = NUM_BUCKETS // 2
    buckets = (d > 0).astype(np.int32) * half
    rp = np.abs(d)
    max_exact = half // 2
    is_small = rp < max_exact
    safe_rp = np.maximum(rp, 1).astype(np.float32)
    large = max_exact + (
        np.log(safe_rp / np.float32(max_exact))
        / np.float32(math.log(MAX_DISTANCE / max_exact))
        * np.float32(half - max_exact)
    ).astype(np.int32)
    large = np.minimum(large, half - 1)
    line = buckets + np.where(is_small, rp, large)

    j = np.arange(LINE_LEN, dtype=np.int32)
    idx = np.minimum(j[None, :] + np.arange(NUM_SHIFTS, dtype=np.int32)[:, None],
                     line.shape[0] - 1)
    return line[idx].astype(np.int32)


_BUCKET_SHIFT = _bucket_shift_table()


def _sc_body(table_hbm, bidx_hbm, out_hbm, table_v, *rest):
    bidx_v = rest[:NUM_SHIFTS]
    lines_v = rest[NUM_SHIFTS:2 * NUM_SHIFTS]
    sem_out = rest[2 * NUM_SHIFTS]

    h = lax.axis_index("s")
    half = lax.axis_index("c")

    pltpu.sync_copy(table_hbm, table_v)
    for s in range(NUM_SHIFTS):
        pltpu.sync_copy(bidx_hbm.at[s], bidx_v[s])

    h_vec = jnp.full((16,), h, dtype=jnp.int32)
    for s in range(NUM_SHIFTS):
        def chunk(i, carry, s=s):
            bv = bidx_v[s][pl.ds(i * 16, 16)]
            lines_v[s][pl.ds(i * 16, 16)] = plsc.load_gather(table_v, [bv, h_vec])
            return carry
        lax.fori_loop(0, CHUNKS, chunk, 0)

    q0 = half * ROWS_PER_WORKER
    def rows(g, carry):
        qb = q0 + g * DMA_GROUP
        copies = []
        for j in range(DMA_GROUP):
            q = qb + j
            s = (Q_LEN - 1 - j) % NUM_SHIFTS
            base = (Q_LEN - 1 - q) - s
            copies.append(pltpu.async_copy(
                lines_v[s].at[pl.ds(base, K_LEN)],
                out_hbm.at[h, q],
                sem_out,
            ))
        @pl.when(g > 0)
        def _drain_prev():
            for c in copies:
                c.wait()
        return carry
    lax.fori_loop(0, ROWS_PER_WORKER // DMA_GROUP, rows, 0)
    for j in range(DMA_GROUP):
        pltpu.make_async_copy(
            lines_v[0].at[pl.ds(0, K_LEN)], out_hbm.at[h, q0], sem_out
        ).wait()


@functools.partial(jax.jit, static_argnums=())
def _bias_sc(bias_table, bucket_shift):
    kern = pl.kernel(
        _sc_body,
        out_type=jax.ShapeDtypeStruct((NUM_HEADS, Q_LEN, K_LEN), jnp.float32),
        mesh=plsc.VectorSubcoreMesh(core_axis_name="c", subcore_axis_name="s"),
        scratch_types=(
            [pltpu.VMEM((NUM_BUCKETS, NUM_HEADS), jnp.float32)]
            + [pltpu.VMEM((LINE_LEN,), jnp.int32) for _ in range(NUM_SHIFTS)]
            + [pltpu.VMEM((LINE_LEN,), jnp.float32) for _ in range(NUM_SHIFTS)]
            + [pltpu.SemaphoreType.DMA]
        ),
        compiler_params=pltpu.CompilerParams(
            needs_layout_passes=False, use_tc_tiling_on_sc=False
        ),
    )
    return kern(bias_table, bucket_shift)


def kernel(query_length, key_length, bias_table):
    del query_length, key_length
    out = _bias_sc(bias_table, jnp.asarray(_BUCKET_SHIFT))
    return out[None]
